# baseline (speedup 1.0000x reference)
import functools

import jax
import jax.numpy as jnp
from jax import lax
from jax.experimental import pallas as pl
from jax.experimental.pallas import tpu as pltpu
from jax.experimental.pallas import tpu_sc as plsc

N = 10000
D = 128
H = 128
HD = D // 2
NC = 2
NS = 16
NW = NC * NS
CH = 128
NPAD = 10240
PADROWS = NPAD - N
ROWS_PER_SUB = NPAD // NS


def _mesh():
    return plsc.VectorSubcoreMesh(core_axis_name="c", subcore_axis_name="s",
                                  num_cores=NC, num_subcores=NS)


def _sc_degrees(nch):
    nch_c = nch // 2

    @functools.partial(
        pl.kernel,
        out_type=jax.ShapeDtypeStruct((NC, 2, NPAD, 16), jnp.float32),
        mesh=_mesh(),
        scratch_types=[
            pltpu.VMEM((nch_c, CH), jnp.int32),
            pltpu.VMEM((nch_c, CH), jnp.int32),
            pltpu.VMEM((CH, 16), jnp.float32),
            pltpu.VMEM_SHARED((NPAD, 16), jnp.float32),
            pltpu.VMEM_SHARED((NPAD, 16), jnp.float32),
            pltpu.SemaphoreType.DMA,
            pltpu.SemaphoreType.DMA,
        ],
    )
    def deg_kernel(src_hbm, dst_hbm, ones_hbm, zeros_hbm, out_hbm,
                   sidx, didx, ones_v, dsrc_sh, ddst_sh, sem_a, sem_b):
        cid = lax.axis_index("c")
        sid = lax.axis_index("s")
        rbase = sid * ROWS_PER_SUB
        pltpu.sync_copy(zeros_hbm.at[pl.ds(rbase, ROWS_PER_SUB)],
                        dsrc_sh.at[pl.ds(rbase, ROWS_PER_SUB)])
        pltpu.sync_copy(zeros_hbm.at[pl.ds(rbase, ROWS_PER_SUB)],
                        ddst_sh.at[pl.ds(rbase, ROWS_PER_SUB)])
        pltpu.sync_copy(ones_hbm, ones_v)
        pltpu.sync_copy(src_hbm.at[sid, pl.ds(cid * nch_c, nch_c)], sidx)
        pltpu.sync_copy(dst_hbm.at[sid, pl.ds(cid * nch_c, nch_c)], didx)
        plsc.subcore_barrier()

        pltpu.async_copy(ones_v, dsrc_sh.at[sidx.at[0]], sem_a, add=True)
        pltpu.async_copy(ones_v, ddst_sh.at[didx.at[0]], sem_b, add=True)

        @pl.loop(1, nch_c)
        def _(j):
            pltpu.async_copy(ones_v, dsrc_sh.at[sidx.at[j]], sem_a, add=True)
            pltpu.async_copy(ones_v, ddst_sh.at[didx.at[j]], sem_b, add=True)
            pltpu.make_async_copy(ones_v, dsrc_sh.at[sidx.at[j]], sem_a).wait()
            pltpu.make_async_copy(ones_v, ddst_sh.at[didx.at[j]], sem_b).wait()

        pltpu.make_async_copy(ones_v, dsrc_sh.at[sidx.at[0]], sem_a).wait()
        pltpu.make_async_copy(ones_v, ddst_sh.at[didx.at[0]], sem_b).wait()
        plsc.subcore_barrier()
        pltpu.sync_copy(dsrc_sh.at[pl.ds(rbase, ROWS_PER_SUB)],
                        out_hbm.at[cid, 0, pl.ds(rbase, ROWS_PER_SUB)])
        pltpu.sync_copy(ddst_sh.at[pl.ds(rbase, ROWS_PER_SUB)],
                        out_hbm.at[cid, 1, pl.ds(rbase, ROWS_PER_SUB)])

    return deg_kernel


def _sc_aggregate(nch):
    assert nch % 2 == 0

    @functools.partial(
        pl.kernel,
        out_type=jax.ShapeDtypeStruct((NC, NPAD, HD), jnp.float32),
        mesh=_mesh(),
        scratch_types=[
            pltpu.VMEM((2 * nch, CH), jnp.int32),
            pltpu.VMEM((CH, HD), jnp.float32),
            pltpu.VMEM((CH, HD), jnp.float32),
            pltpu.VMEM_SHARED((NPAD, HD), jnp.float32),
            pltpu.SemaphoreType.DMA,
            pltpu.SemaphoreType.DMA,
            pltpu.SemaphoreType.DMA,
            pltpu.SemaphoreType.DMA,
        ],
        compiler_params=pltpu.CompilerParams(use_tc_tiling_on_sc=False),
    )
    def agg_kernel(t_hbm, eidx_hbm, zeros_hbm, out_hbm,
                   cidx, rows0, rows1, agg_sh, semg0, semg1, sems0, sems1):
        cid = lax.axis_index("c")
        sid = lax.axis_index("s")
        rbase = sid * ROWS_PER_SUB
        pltpu.sync_copy(zeros_hbm.at[pl.ds(rbase, ROWS_PER_SUB)],
                        agg_sh.at[pl.ds(rbase, ROWS_PER_SUB)])
        pltpu.sync_copy(eidx_hbm.at[sid], cidx)
        plsc.subcore_barrier()

        rows = (rows0, rows1)
        semg = (semg0, semg1)
        sems = (sems0, sems1)

        def g_start(k, p):
            pltpu.async_copy(t_hbm.at[cid].at[cidx.at[2 * k]], rows[p],
                             semg[p])

        def g_wait(k, p):
            pltpu.make_async_copy(t_hbm.at[cid].at[cidx.at[2 * k]],
                                  rows[p], semg[p]).wait()

        def s_start(k, p):
            pltpu.async_copy(rows[p], agg_sh.at[cidx.at[2 * k + 1]],
                             sems[p], add=True)

        def s_wait(k, p):
            pltpu.make_async_copy(rows[p], agg_sh.at[cidx.at[2 * k + 1]],
                                  sems[p]).wait()

        g_start(0, 0)

        @pl.loop(0, nch, step=2)
        def _(j):
            g_wait(j, 0)
            s_start(j, 0)

            @pl.when(j > 0)
            def _():
                s_wait(j - 1, 1)

            g_start(j + 1, 1)
            g_wait(j + 1, 1)
            s_start(j + 1, 1)
            s_wait(j, 0)

            @pl.when(j + 2 < nch)
            def _():
                g_start(j + 2, 0)

        s_wait(nch - 1, 1)
        plsc.subcore_barrier()
        pltpu.sync_copy(agg_sh.at[pl.ds(rbase, ROWS_PER_SUB)],
                        out_hbm.at[cid, pl.ds(rbase, ROWS_PER_SUB)])

    return agg_kernel


def _tc_prep(degp_ref, x_ref, xs_ref, ns_ref, nd_ref):
    dsrc = degp_ref[0, 0, :, 0:1] + degp_ref[1, 0, :, 0:1]
    ddst = degp_ref[0, 1, :, 0:1] + degp_ref[1, 1, :, 0:1]
    ns = lax.rsqrt(jnp.maximum(dsrc, 1.0))
    nd = lax.rsqrt(jnp.maximum(ddst, 1.0))
    ns_ref[...] = ns
    nd_ref[...] = nd
    xs = x_ref[...] * ns
    xs_ref[0] = xs[:, :HD]
    xs_ref[1] = xs[:, HD:]


def _layer_body(p_ref, h_ref, nd_ref, w_ref, b_ref, wr_ref, br_ref):
    agg = jnp.concatenate([p_ref[0], p_ref[1]], axis=1) * nd_ref[...]
    conv = jnp.maximum(
        jnp.dot(agg, w_ref[...], preferred_element_type=jnp.float32)
        + b_ref[...], 0.0)
    res = jnp.maximum(
        jnp.dot(h_ref[...], wr_ref[...], preferred_element_type=jnp.float32)
        + br_ref[...], 0.0)
    h = conv + res
    rid = lax.broadcasted_iota(jnp.int32, (NPAD, 1), 0)
    return jnp.where(rid < N, h, 0.0)


def _tc_layer(p_ref, h_ref, ns_ref, nd_ref, w_ref, b_ref, wr_ref, br_ref,
              out_h_ref, hs_ref):
    h = _layer_body(p_ref, h_ref, nd_ref, w_ref, b_ref, wr_ref, br_ref)
    out_h_ref[...] = h
    hs = h * ns_ref[...]
    hs_ref[0] = hs[:, :HD]
    hs_ref[1] = hs[:, HD:]


def _tc_final(p_ref, h_ref, nd_ref, w_ref, b_ref, wr_ref, br_ref,
              wg_ref, bg_ref, gamma_ref, beta_ref, wc1_ref, bc1_ref,
              wc2_ref, bc2_ref, out_ref):
    h = _layer_body(p_ref, h_ref, nd_ref, w_ref, b_ref, wr_ref, br_ref)
    logit = jnp.dot(h, wg_ref[...], preferred_element_type=jnp.float32) \
        + bg_ref[...]
    w = 1.0 / (1.0 + jnp.exp(-logit))
    gsum = jnp.sum(h * w, axis=0)
    gmax = jnp.max(h, axis=0)
    g = jnp.concatenate([gsum, gmax])[None, :]
    z = jnp.maximum(
        jnp.dot(g, wc1_ref[...], preferred_element_type=jnp.float32)
        + bc1_ref[...], 0.0)
    z = (z * float(1.0 / (1.0 + 1e-5) ** 0.5)) * gamma_ref[...] + beta_ref[...]
    zo = jnp.dot(z, wc2_ref[...], preferred_element_type=jnp.float32) \
        + bc2_ref[...]
    out_ref[...] = 1.0 / (1.0 + jnp.exp(-zo))


def _tc_call(fn, out_shapes):
    return pl.pallas_call(fn, out_shape=out_shapes)


def kernel(x, edge_index, W0, b0, Wres0, bres0, W1, b1, Wres1, bres1,
           Wg, bg, gamma, beta, Wc1, bc1, Wc2, bc2):
    e = edge_index.shape[1]
    step = NS * CH
    nch = (e + step - 1) // step
    nch = ((nch + 15) // 16) * 16
    epad = nch * step

    src = edge_index[0].astype(jnp.int32)
    dst = edge_index[1].astype(jnp.int32)
    pad_idx = (N + jnp.arange(epad - e, dtype=jnp.int32) % PADROWS)
    src3 = jnp.concatenate([src, pad_idx]).reshape(NS, nch, CH)
    dst3 = jnp.concatenate([dst, pad_idx]).reshape(NS, nch, CH)
    eidx = jnp.stack([src3, dst3], axis=2).reshape(NS, 2 * nch, CH)

    x_pad = jnp.zeros((NPAD, D), jnp.float32).at[:N].set(x)
    zeros_nd = jnp.zeros((NPAD, HD), jnp.float32)
    zeros_16 = jnp.zeros((NPAD, 16), jnp.float32)
    ones_rows = jnp.ones((CH, 16), jnp.float32)

    degp = _sc_degrees(nch)(src3, dst3, ones_rows, zeros_16)

    f32 = jnp.float32
    xs, ns, nd = _tc_call(_tc_prep, [
        jax.ShapeDtypeStruct((NC, NPAD, HD), f32),
        jax.ShapeDtypeStruct((NPAD, 1), f32),
        jax.ShapeDtypeStruct((NPAD, 1), f32),
    ])(degp, x_pad)

    agg_fn = _sc_aggregate(nch)
    p0 = agg_fn(xs, eidx, zeros_nd)

    h1, h1s = _tc_call(_tc_layer, [
        jax.ShapeDtypeStruct((NPAD, H), f32),
        jax.ShapeDtypeStruct((NC, NPAD, HD), f32),
    ])(p0, x_pad, ns, nd, W0, b0[None, :], Wres0, bres0[None, :])

    p1 = agg_fn(h1s, eidx, zeros_nd)

    (out,) = _tc_call(_tc_final, [jax.ShapeDtypeStruct((1, 1), f32)])(
        p1, h1, nd, W1, b1[None, :], Wres1, bres1[None, :],
        Wg, bg[None, :], gamma[None, :], beta[None, :],
        Wc1, bc1[None, :], Wc2, bc2[None, :])
    return out


# speedup vs baseline: 1.2674x; 1.2674x over previous
import functools

import jax
import jax.numpy as jnp
from jax import lax
from jax.experimental import pallas as pl
from jax.experimental.pallas import tpu as pltpu
from jax.experimental.pallas import tpu_sc as plsc

N = 10000
D = 128
H = 128
NC = 2
NS = 16
NW = NC * NS
CH = 128
NPAD = 10240
PADROWS = NPAD - N
ROWS_PER_SUB = NPAD // NS


def _mesh():
    return plsc.VectorSubcoreMesh(core_axis_name="c", subcore_axis_name="s",
                                  num_cores=NC, num_subcores=NS)


def _sc_degrees(nch):
    @functools.partial(
        pl.kernel,
        out_type=jax.ShapeDtypeStruct((NC, 2, NPAD, 16), jnp.float32),
        mesh=_mesh(),
        scratch_types=[
            pltpu.VMEM((2 * nch, CH), jnp.int32),
            pltpu.VMEM((CH, 16), jnp.float32),
            pltpu.VMEM_SHARED((NPAD, 16), jnp.float32),
            pltpu.VMEM_SHARED((NPAD, 16), jnp.float32),
            pltpu.SemaphoreType.DMA,
            pltpu.SemaphoreType.DMA,
        ],
    )
    def deg_kernel(eidx_hbm, ones_hbm, zeros_hbm, out_hbm,
                   cidx, ones_v, dsrc_sh, ddst_sh, sem_a, sem_b):
        cid = lax.axis_index("c")
        sid = lax.axis_index("s")
        wid = sid * NC + cid
        rbase = sid * ROWS_PER_SUB
        pltpu.sync_copy(zeros_hbm.at[pl.ds(rbase, ROWS_PER_SUB)],
                        dsrc_sh.at[pl.ds(rbase, ROWS_PER_SUB)])
        pltpu.sync_copy(zeros_hbm.at[pl.ds(rbase, ROWS_PER_SUB)],
                        ddst_sh.at[pl.ds(rbase, ROWS_PER_SUB)])
        pltpu.sync_copy(ones_hbm, ones_v)
        pltpu.sync_copy(eidx_hbm.at[wid], cidx)
        plsc.subcore_barrier()

        pltpu.async_copy(ones_v, dsrc_sh.at[cidx.at[0]], sem_a, add=True)
        pltpu.async_copy(ones_v, ddst_sh.at[cidx.at[1]], sem_b, add=True)

        @pl.loop(1, nch)
        def _(j):
            pltpu.async_copy(ones_v, dsrc_sh.at[cidx.at[2 * j]], sem_a,
                             add=True)
            pltpu.async_copy(ones_v, ddst_sh.at[cidx.at[2 * j + 1]], sem_b,
                             add=True)
            pltpu.make_async_copy(ones_v, dsrc_sh.at[cidx.at[2 * j]],
                                  sem_a).wait()
            pltpu.make_async_copy(ones_v, ddst_sh.at[cidx.at[2 * j + 1]],
                                  sem_b).wait()

        pltpu.make_async_copy(ones_v, dsrc_sh.at[cidx.at[0]], sem_a).wait()
        pltpu.make_async_copy(ones_v, ddst_sh.at[cidx.at[1]], sem_b).wait()
        plsc.subcore_barrier()
        pltpu.sync_copy(dsrc_sh.at[pl.ds(rbase, ROWS_PER_SUB)],
                        out_hbm.at[cid, 0, pl.ds(rbase, ROWS_PER_SUB)])
        pltpu.sync_copy(ddst_sh.at[pl.ds(rbase, ROWS_PER_SUB)],
                        out_hbm.at[cid, 1, pl.ds(rbase, ROWS_PER_SUB)])

    return deg_kernel


def _sc_aggregate(nch, nblk, blk):
    assert blk % 2 == 0

    @functools.partial(
        pl.kernel,
        out_type=jax.ShapeDtypeStruct((NC, NPAD, D), jnp.float32),
        mesh=_mesh(),
        scratch_types=[
            pltpu.VMEM((2 * blk, CH), jnp.int32),
            pltpu.VMEM((CH, D), jnp.float32),
            pltpu.VMEM((CH, D), jnp.float32),
            pltpu.VMEM_SHARED((NPAD, D), jnp.float32),
            pltpu.SemaphoreType.DMA,
            pltpu.SemaphoreType.DMA,
            pltpu.SemaphoreType.DMA,
            pltpu.SemaphoreType.DMA,
        ],
    )
    def agg_kernel(t_hbm, eidx_hbm, zeros_hbm, out_hbm,
                   cidx, rows0, rows1, agg_sh, semg0, semg1, sems0, sems1):
        cid = lax.axis_index("c")
        sid = lax.axis_index("s")
        wid = sid * NC + cid
        rbase = sid * ROWS_PER_SUB
        pltpu.sync_copy(zeros_hbm.at[pl.ds(rbase, ROWS_PER_SUB)],
                        agg_sh.at[pl.ds(rbase, ROWS_PER_SUB)])
        plsc.subcore_barrier()

        rows = (rows0, rows1)
        semg = (semg0, semg1)
        sems = (sems0, sems1)

        def g_start(k):
            pltpu.async_copy(t_hbm.at[cidx.at[2 * k]], rows[k % 2],
                             semg[k % 2])

        def g_wait(k):
            pltpu.make_async_copy(t_hbm.at[cidx.at[2 * k]], rows[k % 2],
                                  semg[k % 2]).wait()

        def s_start(k):
            pltpu.async_copy(rows[k % 2], agg_sh.at[cidx.at[2 * k + 1]],
                             sems[k % 2], add=True)

        def s_wait(k):
            pltpu.make_async_copy(rows[k % 2], agg_sh.at[cidx.at[2 * k + 1]],
                                  sems[k % 2]).wait()

        @pl.loop(0, nblk)
        def _(b):
            off = pl.multiple_of(2 * b * blk, 16)
            pltpu.sync_copy(eidx_hbm.at[wid, pl.ds(off, 2 * blk)], cidx)
            g_start(0)
            for k in range(blk):
                if k == 0:
                    @pl.when(b > 0)
                    def _():
                        s_wait(blk - 1)
                g_wait(k)
                s_start(k)
                if k >= 1:
                    s_wait(k - 1)
                if k + 1 < blk:
                    g_start(k + 1)

        s_wait(blk - 1)
        plsc.subcore_barrier()
        pltpu.sync_copy(agg_sh.at[pl.ds(rbase, ROWS_PER_SUB)],
                        out_hbm.at[cid, pl.ds(rbase, ROWS_PER_SUB)])

    return agg_kernel


def _tc_prep(degp_ref, x_ref, xs_ref, ns_ref, nd_ref):
    dsrc = degp_ref[0, 0, :, 0:1] + degp_ref[1, 0, :, 0:1]
    ddst = degp_ref[0, 1, :, 0:1] + degp_ref[1, 1, :, 0:1]
    ns = lax.rsqrt(jnp.maximum(dsrc, 1.0))
    nd = lax.rsqrt(jnp.maximum(ddst, 1.0))
    ns_ref[...] = ns
    nd_ref[...] = nd
    xs_ref[pl.ds(0, N), :] = x_ref[...] * ns[:N]
    xs_ref[pl.ds(N, PADROWS), :] = jnp.zeros((PADROWS, D), jnp.float32)


def _layer_body(p_ref, h, nd_ref, w_ref, b_ref, wr_ref, br_ref):
    agg = (p_ref[0] + p_ref[1]) * nd_ref[...]
    conv = jnp.maximum(
        jnp.dot(agg, w_ref[...], preferred_element_type=jnp.float32)
        + b_ref[...], 0.0)
    res = jnp.maximum(
        jnp.dot(h, wr_ref[...], preferred_element_type=jnp.float32)
        + br_ref[...], 0.0)
    h = conv + res
    rid = lax.broadcasted_iota(jnp.int32, (NPAD, 1), 0)
    return jnp.where(rid < N, h, 0.0)


def _tc_layer(p_ref, x_ref, ns_ref, nd_ref, w_ref, b_ref, wr_ref, br_ref,
              out_h_ref, hs_ref):
    x = jnp.concatenate([x_ref[...], jnp.zeros((PADROWS, D), jnp.float32)])
    h = _layer_body(p_ref, x, nd_ref, w_ref, b_ref, wr_ref, br_ref)
    out_h_ref[...] = h
    hs_ref[...] = h * ns_ref[...]


def _tc_final(p_ref, h_ref, nd_ref, w_ref, b_ref, wr_ref, br_ref,
              wg_ref, bg_ref, gamma_ref, beta_ref, wc1_ref, bc1_ref,
              wc2_ref, bc2_ref, out_ref):
    h = _layer_body(p_ref, h_ref[...], nd_ref, w_ref, b_ref, wr_ref, br_ref)
    logit = jnp.dot(h, wg_ref[...], preferred_element_type=jnp.float32) \
        + bg_ref[...]
    w = 1.0 / (1.0 + jnp.exp(-logit))
    gsum = jnp.sum(h * w, axis=0)
    gmax = jnp.max(h, axis=0)
    g = jnp.concatenate([gsum, gmax])[None, :]
    z = jnp.maximum(
        jnp.dot(g, wc1_ref[...], preferred_element_type=jnp.float32)
        + bc1_ref[...], 0.0)
    z = (z * float(1.0 / (1.0 + 1e-5) ** 0.5)) * gamma_ref[...] + beta_ref[...]
    zo = jnp.dot(z, wc2_ref[...], preferred_element_type=jnp.float32) \
        + bc2_ref[...]
    out_ref[...] = 1.0 / (1.0 + jnp.exp(-zo))


def _tc_call(fn, out_shapes):
    return pl.pallas_call(fn, out_shape=out_shapes)


def kernel(x, edge_index, W0, b0, Wres0, bres0, W1, b1, Wres1, bres1,
           Wg, bg, gamma, beta, Wc1, bc1, Wc2, bc2):
    e = edge_index.shape[1]
    blk = 8
    step = NW * CH
    nch = (e + step - 1) // step
    nch = ((nch + 2 * blk - 1) // (2 * blk)) * (2 * blk)
    nblk = nch // blk
    epad = nch * step

    src = edge_index[0].astype(jnp.int32)
    dst = edge_index[1].astype(jnp.int32)
    pad_idx = (N + jnp.arange(epad - e, dtype=jnp.int32) % PADROWS)
    src3 = jnp.concatenate([src, pad_idx]).reshape(NW, nch, CH)
    dst3 = jnp.concatenate([dst, pad_idx]).reshape(NW, nch, CH)
    eidx = jnp.stack([src3, dst3], axis=2).reshape(NW, 2 * nch, CH)

    zeros_nd = jnp.zeros((NPAD, D), jnp.float32)
    zeros_16 = jnp.zeros((NPAD, 16), jnp.float32)
    ones_rows = jnp.ones((CH, 16), jnp.float32)

    degp = _sc_degrees(nch)(eidx, ones_rows, zeros_16)

    f32 = jnp.float32
    xs, ns, nd = _tc_call(_tc_prep, [
        jax.ShapeDtypeStruct((NPAD, D), f32),
        jax.ShapeDtypeStruct((NPAD, 1), f32),
        jax.ShapeDtypeStruct((NPAD, 1), f32),
    ])(degp, x)

    agg_fn = _sc_aggregate(nch, nblk, blk)
    p0 = agg_fn(xs, eidx, zeros_nd)

    h1, h1s = _tc_call(_tc_layer, [
        jax.ShapeDtypeStruct((NPAD, H), f32),
        jax.ShapeDtypeStruct((NPAD, H), f32),
    ])(p0, x, ns, nd, W0, b0[None, :], Wres0, bres0[None, :])

    p1 = agg_fn(h1s, eidx, zeros_nd)

    (out,) = _tc_call(_tc_final, [jax.ShapeDtypeStruct((1, 1), f32)])(
        p1, h1, nd, W1, b1[None, :], Wres1, bres1[None, :],
        Wg, bg[None, :], gamma[None, :], beta[None, :],
        Wc1, bc1[None, :], Wc2, bc2[None, :])
    return out


# speedup vs baseline: 1.2792x; 1.0093x over previous
import functools

import jax
import jax.numpy as jnp
from jax import lax
from jax.experimental import pallas as pl
from jax.experimental.pallas import tpu as pltpu
from jax.experimental.pallas import tpu_sc as plsc

N = 10000
D = 128
H = 128
NC = 2
NS = 16
NW = NC * NS
CH = 128
NPAD = 10240
PADROWS = NPAD - N
ROWS_PER_SUB = NPAD // NS


def _mesh():
    return plsc.VectorSubcoreMesh(core_axis_name="c", subcore_axis_name="s",
                                  num_cores=NC, num_subcores=NS)


def _sc_degrees(nch):
    @functools.partial(
        pl.kernel,
        out_type=jax.ShapeDtypeStruct((NC, 2, NPAD, 16), jnp.float32),
        mesh=_mesh(),
        scratch_types=[
            pltpu.VMEM((2 * nch, CH), jnp.int32),
            pltpu.VMEM((CH, 16), jnp.float32),
            pltpu.VMEM_SHARED((NPAD, 16), jnp.float32),
            pltpu.VMEM_SHARED((NPAD, 16), jnp.float32),
            pltpu.SemaphoreType.DMA,
            pltpu.SemaphoreType.DMA,
        ],
    )
    def deg_kernel(eidx_hbm, ones_hbm, zeros_hbm, out_hbm,
                   cidx, ones_v, dsrc_sh, ddst_sh, sem_a, sem_b):
        cid = lax.axis_index("c")
        sid = lax.axis_index("s")
        wid = sid * NC + cid
        rbase = sid * ROWS_PER_SUB
        pltpu.sync_copy(zeros_hbm.at[pl.ds(rbase, ROWS_PER_SUB)],
                        dsrc_sh.at[pl.ds(rbase, ROWS_PER_SUB)])
        pltpu.sync_copy(zeros_hbm.at[pl.ds(rbase, ROWS_PER_SUB)],
                        ddst_sh.at[pl.ds(rbase, ROWS_PER_SUB)])
        pltpu.sync_copy(ones_hbm, ones_v)
        pltpu.sync_copy(eidx_hbm.at[wid], cidx)
        plsc.subcore_barrier()

        pltpu.async_copy(ones_v, dsrc_sh.at[cidx.at[0]], sem_a, add=True)
        pltpu.async_copy(ones_v, ddst_sh.at[cidx.at[1]], sem_b, add=True)

        @pl.loop(1, nch)
        def _(j):
            pltpu.async_copy(ones_v, dsrc_sh.at[cidx.at[2 * j]], sem_a,
                             add=True)
            pltpu.async_copy(ones_v, ddst_sh.at[cidx.at[2 * j + 1]], sem_b,
                             add=True)
            pltpu.make_async_copy(ones_v, dsrc_sh.at[cidx.at[2 * j]],
                                  sem_a).wait()
            pltpu.make_async_copy(ones_v, ddst_sh.at[cidx.at[2 * j + 1]],
                                  sem_b).wait()

        pltpu.make_async_copy(ones_v, dsrc_sh.at[cidx.at[0]], sem_a).wait()
        pltpu.make_async_copy(ones_v, ddst_sh.at[cidx.at[1]], sem_b).wait()
        plsc.subcore_barrier()
        pltpu.sync_copy(dsrc_sh.at[pl.ds(rbase, ROWS_PER_SUB)],
                        out_hbm.at[cid, 0, pl.ds(rbase, ROWS_PER_SUB)])
        pltpu.sync_copy(ddst_sh.at[pl.ds(rbase, ROWS_PER_SUB)],
                        out_hbm.at[cid, 1, pl.ds(rbase, ROWS_PER_SUB)])

    return deg_kernel


def _sc_aggregate(nch, nblk, blk):
    assert blk % 2 == 0

    @functools.partial(
        pl.kernel,
        out_type=jax.ShapeDtypeStruct((NC, NPAD, D), jnp.float32),
        mesh=_mesh(),
        scratch_types=[
            pltpu.VMEM((2 * blk, CH), jnp.int32),
            pltpu.VMEM((CH, D), jnp.float32),
            pltpu.VMEM((CH, D), jnp.float32),
            pltpu.VMEM_SHARED((NPAD, D), jnp.float32),
            pltpu.SemaphoreType.DMA,
            pltpu.SemaphoreType.DMA,
            pltpu.SemaphoreType.DMA,
            pltpu.SemaphoreType.DMA,
            pltpu.SemaphoreType.DMA,
        ],
    )
    def agg_kernel(t_hbm, eidx_hbm, zeros_hbm, out_hbm,
                   cidx, rows0, rows1, agg_sh, semg0, semg1, sems0, sems1,
                   semz):
        cid = lax.axis_index("c")
        sid = lax.axis_index("s")
        wid = sid * NC + cid
        rbase = sid * ROWS_PER_SUB
        pltpu.async_copy(zeros_hbm.at[pl.ds(rbase, ROWS_PER_SUB)],
                        agg_sh.at[pl.ds(rbase, ROWS_PER_SUB)], semz)

        rows = (rows0, rows1)
        semg = (semg0, semg1)
        sems = (sems0, sems1)

        def g_start(k):
            pltpu.async_copy(t_hbm.at[cidx.at[2 * k]], rows[k % 2],
                             semg[k % 2])

        def g_wait(k):
            pltpu.make_async_copy(t_hbm.at[cidx.at[2 * k]], rows[k % 2],
                                  semg[k % 2]).wait()

        def s_start(k):
            pltpu.async_copy(rows[k % 2], agg_sh.at[cidx.at[2 * k + 1]],
                             sems[k % 2], add=True)

        def s_wait(k):
            pltpu.make_async_copy(rows[k % 2], agg_sh.at[cidx.at[2 * k + 1]],
                                  sems[k % 2]).wait()

        @pl.loop(0, nblk)
        def _(b):
            off = pl.multiple_of(2 * b * blk, 16)
            pltpu.sync_copy(eidx_hbm.at[wid, pl.ds(off, 2 * blk)], cidx)
            g_start(0)
            for k in range(blk):
                if k == 0:
                    @pl.when(b == 0)
                    def _():
                        pltpu.make_async_copy(
                            zeros_hbm.at[pl.ds(rbase, ROWS_PER_SUB)],
                            agg_sh.at[pl.ds(rbase, ROWS_PER_SUB)],
                            semz).wait()
                        plsc.subcore_barrier()

                    @pl.when(b > 0)
                    def _():
                        s_wait(blk - 1)
                g_wait(k)
                s_start(k)
                if k >= 1:
                    s_wait(k - 1)
                if k + 1 < blk:
                    g_start(k + 1)

        s_wait(blk - 1)
        plsc.subcore_barrier()
        pltpu.sync_copy(agg_sh.at[pl.ds(rbase, ROWS_PER_SUB)],
                        out_hbm.at[cid, pl.ds(rbase, ROWS_PER_SUB)])

    return agg_kernel


def _tc_prep(degp_ref, x_ref, xs_ref, ns_ref, nd_ref):
    dsrc = degp_ref[0, 0, :, 0:1] + degp_ref[1, 0, :, 0:1]
    ddst = degp_ref[0, 1, :, 0:1] + degp_ref[1, 1, :, 0:1]
    ns = lax.rsqrt(jnp.maximum(dsrc, 1.0))
    nd = lax.rsqrt(jnp.maximum(ddst, 1.0))
    ns_ref[...] = ns
    nd_ref[...] = nd
    xs_ref[pl.ds(0, N), :] = x_ref[...] * ns[:N]
    xs_ref[pl.ds(N, PADROWS), :] = jnp.zeros((PADROWS, D), jnp.float32)


def _layer_body(p_ref, h, nd_ref, w_ref, b_ref, wr_ref, br_ref):
    agg = (p_ref[0] + p_ref[1]) * nd_ref[...]
    conv = jnp.maximum(
        jnp.dot(agg, w_ref[...], preferred_element_type=jnp.float32)
        + b_ref[...], 0.0)
    res = jnp.maximum(
        jnp.dot(h, wr_ref[...], preferred_element_type=jnp.float32)
        + br_ref[...], 0.0)
    h = conv + res
    rid = lax.broadcasted_iota(jnp.int32, (NPAD, 1), 0)
    return jnp.where(rid < N, h, 0.0)


def _tc_layer(p_ref, x_ref, ns_ref, nd_ref, w_ref, b_ref, wr_ref, br_ref,
              out_h_ref, hs_ref):
    x = jnp.concatenate([x_ref[...], jnp.zeros((PADROWS, D), jnp.float32)])
    h = _layer_body(p_ref, x, nd_ref, w_ref, b_ref, wr_ref, br_ref)
    out_h_ref[...] = h
    hs_ref[...] = h * ns_ref[...]


def _tc_final(p_ref, h_ref, nd_ref, w_ref, b_ref, wr_ref, br_ref,
              wg_ref, bg_ref, gamma_ref, beta_ref, wc1_ref, bc1_ref,
              wc2_ref, bc2_ref, out_ref):
    h = _layer_body(p_ref, h_ref[...], nd_ref, w_ref, b_ref, wr_ref, br_ref)
    logit = jnp.dot(h, wg_ref[...], preferred_element_type=jnp.float32) \
        + bg_ref[...]
    w = 1.0 / (1.0 + jnp.exp(-logit))
    gsum = jnp.sum(h * w, axis=0)
    gmax = jnp.max(h, axis=0)
    g = jnp.concatenate([gsum, gmax])[None, :]
    z = jnp.maximum(
        jnp.dot(g, wc1_ref[...], preferred_element_type=jnp.float32)
        + bc1_ref[...], 0.0)
    z = (z * float(1.0 / (1.0 + 1e-5) ** 0.5)) * gamma_ref[...] + beta_ref[...]
    zo = jnp.dot(z, wc2_ref[...], preferred_element_type=jnp.float32) \
        + bc2_ref[...]
    out_ref[...] = 1.0 / (1.0 + jnp.exp(-zo))


def _tc_call(fn, out_shapes):
    return pl.pallas_call(fn, out_shape=out_shapes)


def kernel(x, edge_index, W0, b0, Wres0, bres0, W1, b1, Wres1, bres1,
           Wg, bg, gamma, beta, Wc1, bc1, Wc2, bc2):
    e = edge_index.shape[1]
    blk = 8
    step = NW * CH
    nch = (e + step - 1) // step
    nch = ((nch + 2 * blk - 1) // (2 * blk)) * (2 * blk)
    nblk = nch // blk
    epad = nch * step

    src = edge_index[0].astype(jnp.int32)
    dst = edge_index[1].astype(jnp.int32)
    pad_idx = (N + jnp.arange(epad - e, dtype=jnp.int32) % PADROWS)
    src3 = jnp.concatenate([src, pad_idx]).reshape(NW, nch, CH)
    dst3 = jnp.concatenate([dst, pad_idx]).reshape(NW, nch, CH)
    eidx = jnp.stack([src3, dst3], axis=2).reshape(NW, 2 * nch, CH)

    zeros_nd = jnp.zeros((NPAD, D), jnp.float32)
    zeros_16 = jnp.zeros((NPAD, 16), jnp.float32)
    ones_rows = jnp.ones((CH, 16), jnp.float32)

    degp = _sc_degrees(nch)(eidx, ones_rows, zeros_16)

    f32 = jnp.float32
    xs, ns, nd = _tc_call(_tc_prep, [
        jax.ShapeDtypeStruct((NPAD, D), f32),
        jax.ShapeDtypeStruct((NPAD, 1), f32),
        jax.ShapeDtypeStruct((NPAD, 1), f32),
    ])(degp, x)

    agg_fn = _sc_aggregate(nch, nblk, blk)
    p0 = agg_fn(xs, eidx, zeros_nd)

    h1, h1s = _tc_call(_tc_layer, [
        jax.ShapeDtypeStruct((NPAD, H), f32),
        jax.ShapeDtypeStruct((NPAD, H), f32),
    ])(p0, x, ns, nd, W0, b0[None, :], Wres0, bres0[None, :])

    p1 = agg_fn(h1s, eidx, zeros_nd)

    (out,) = _tc_call(_tc_final, [jax.ShapeDtypeStruct((1, 1), f32)])(
        p1, h1, nd, W1, b1[None, :], Wres1, bres1[None, :],
        Wg, bg[None, :], gamma[None, :], beta[None, :],
        Wc1, bc1[None, :], Wc2, bc2[None, :])
    return out


# speedup vs baseline: 1.3489x; 1.0545x over previous
import functools

import jax
import jax.numpy as jnp
from jax import lax
from jax.experimental import pallas as pl
from jax.experimental.pallas import tpu as pltpu
from jax.experimental.pallas import tpu_sc as plsc

N = 10000
D = 128
H = 128
NC = 2
NS = 16
NW = NC * NS
CH = 128
NPAD = 10240
PADROWS = NPAD - N
ROWS_PER_SUB = NPAD // NS


def _mesh():
    return plsc.VectorSubcoreMesh(core_axis_name="c", subcore_axis_name="s",
                                  num_cores=NC, num_subcores=NS)


def _sc_degrees(nch):
    @functools.partial(
        pl.kernel,
        out_type=jax.ShapeDtypeStruct((NW, 2, NPAD), jnp.float32),
        mesh=_mesh(),
        scratch_types=[
            pltpu.VMEM((2 * nch, CH), jnp.int32),
            pltpu.VMEM((NPAD,), jnp.float32),
            pltpu.VMEM((NPAD,), jnp.float32),
        ],
        compiler_params=pltpu.CompilerParams(needs_layout_passes=False),
    )
    def deg_kernel(eidx_hbm, zeros_hbm, out_hbm, cidx, dsrc_v, ddst_v):
        cid = lax.axis_index("c")
        sid = lax.axis_index("s")
        wid = sid * NC + cid
        pltpu.sync_copy(zeros_hbm, dsrc_v)
        pltpu.sync_copy(zeros_hbm, ddst_v)
        pltpu.sync_copy(eidx_hbm.at[wid], cidx)

        @pl.loop(0, nch)
        def _(j):
            for v in range(CH // 16):
                sv = cidx[2 * j, pl.ds(v * 16, 16)]
                cnt, last = plsc.scan_count(sv)
                plsc.addupdate_scatter(dsrc_v, [sv],
                                       cnt.astype(jnp.float32), mask=last)
                dv = cidx[2 * j + 1, pl.ds(v * 16, 16)]
                cnt2, last2 = plsc.scan_count(dv)
                plsc.addupdate_scatter(ddst_v, [dv],
                                       cnt2.astype(jnp.float32), mask=last2)

        pltpu.sync_copy(dsrc_v, out_hbm.at[wid, 0])
        pltpu.sync_copy(ddst_v, out_hbm.at[wid, 1])

    return deg_kernel


def _sc_aggregate(nch, nblk, blk):
    assert blk % 2 == 0

    @functools.partial(
        pl.kernel,
        out_type=jax.ShapeDtypeStruct((NC, NPAD, D), jnp.float32),
        mesh=_mesh(),
        scratch_types=[
            pltpu.VMEM((2 * blk, CH), jnp.int32),
            pltpu.VMEM((CH, D), jnp.float32),
            pltpu.VMEM((CH, D), jnp.float32),
            pltpu.VMEM_SHARED((NPAD, D), jnp.float32),
            pltpu.SemaphoreType.DMA,
            pltpu.SemaphoreType.DMA,
            pltpu.SemaphoreType.DMA,
            pltpu.SemaphoreType.DMA,
            pltpu.SemaphoreType.DMA,
        ],
    )
    def agg_kernel(t_hbm, eidx_hbm, zeros_hbm, out_hbm,
                   cidx, rows0, rows1, agg_sh, semg0, semg1, sems0, sems1,
                   semz):
        cid = lax.axis_index("c")
        sid = lax.axis_index("s")
        wid = sid * NC + cid
        rbase = sid * ROWS_PER_SUB
        pltpu.async_copy(zeros_hbm.at[pl.ds(rbase, ROWS_PER_SUB)],
                        agg_sh.at[pl.ds(rbase, ROWS_PER_SUB)], semz)

        rows = (rows0, rows1)
        semg = (semg0, semg1)
        sems = (sems0, sems1)

        def g_start(k):
            pltpu.async_copy(t_hbm.at[cidx.at[2 * k]], rows[k % 2],
                             semg[k % 2])

        def g_wait(k):
            pltpu.make_async_copy(t_hbm.at[cidx.at[2 * k]], rows[k % 2],
                                  semg[k % 2]).wait()

        def s_start(k):
            pltpu.async_copy(rows[k % 2], agg_sh.at[cidx.at[2 * k + 1]],
                             sems[k % 2], add=True)

        def s_wait(k):
            pltpu.make_async_copy(rows[k % 2], agg_sh.at[cidx.at[2 * k + 1]],
                                  sems[k % 2]).wait()

        @pl.loop(0, nblk)
        def _(b):
            off = pl.multiple_of(2 * b * blk, 16)
            pltpu.sync_copy(eidx_hbm.at[wid, pl.ds(off, 2 * blk)], cidx)
            g_start(0)
            for k in range(blk):
                if k == 0:
                    @pl.when(b == 0)
                    def _():
                        pltpu.make_async_copy(
                            zeros_hbm.at[pl.ds(rbase, ROWS_PER_SUB)],
                            agg_sh.at[pl.ds(rbase, ROWS_PER_SUB)],
                            semz).wait()
                        plsc.subcore_barrier()

                    @pl.when(b > 0)
                    def _():
                        s_wait(blk - 1)
                g_wait(k)
                s_start(k)
                if k >= 1:
                    s_wait(k - 1)
                if k + 1 < blk:
                    g_start(k + 1)

        s_wait(blk - 1)
        plsc.subcore_barrier()
        pltpu.sync_copy(agg_sh.at[pl.ds(rbase, ROWS_PER_SUB)],
                        out_hbm.at[cid, pl.ds(rbase, ROWS_PER_SUB)])

    return agg_kernel


def _tc_prep(degp_ref, x_ref, xs_ref, ns_ref, nd_ref):
    dsrc = jnp.sum(degp_ref[:, 0, :], axis=0)[:, None]
    ddst = jnp.sum(degp_ref[:, 1, :], axis=0)[:, None]
    ns = lax.rsqrt(jnp.maximum(dsrc, 1.0))
    nd = lax.rsqrt(jnp.maximum(ddst, 1.0))
    ns_ref[...] = ns
    nd_ref[...] = nd
    xs_ref[pl.ds(0, N), :] = x_ref[...] * ns[:N]
    xs_ref[pl.ds(N, PADROWS), :] = jnp.zeros((PADROWS, D), jnp.float32)


def _layer_body(p_ref, h, nd_ref, w_ref, b_ref, wr_ref, br_ref):
    agg = (p_ref[0] + p_ref[1]) * nd_ref[...]
    conv = jnp.maximum(
        jnp.dot(agg, w_ref[...], preferred_element_type=jnp.float32)
        + b_ref[...], 0.0)
    res = jnp.maximum(
        jnp.dot(h, wr_ref[...], preferred_element_type=jnp.float32)
        + br_ref[...], 0.0)
    h = conv + res
    rid = lax.broadcasted_iota(jnp.int32, (NPAD, 1), 0)
    return jnp.where(rid < N, h, 0.0)


def _tc_layer(p_ref, x_ref, ns_ref, nd_ref, w_ref, b_ref, wr_ref, br_ref,
              out_h_ref, hs_ref):
    x = jnp.concatenate([x_ref[...], jnp.zeros((PADROWS, D), jnp.float32)])
    h = _layer_body(p_ref, x, nd_ref, w_ref, b_ref, wr_ref, br_ref)
    out_h_ref[...] = h
    hs_ref[...] = h * ns_ref[...]


def _tc_final(p_ref, h_ref, nd_ref, w_ref, b_ref, wr_ref, br_ref,
              wg_ref, bg_ref, gamma_ref, beta_ref, wc1_ref, bc1_ref,
              wc2_ref, bc2_ref, out_ref):
    h = _layer_body(p_ref, h_ref[...], nd_ref, w_ref, b_ref, wr_ref, br_ref)
    logit = jnp.dot(h, wg_ref[...], preferred_element_type=jnp.float32) \
        + bg_ref[...]
    w = 1.0 / (1.0 + jnp.exp(-logit))
    gsum = jnp.sum(h * w, axis=0)
    gmax = jnp.max(h, axis=0)
    g = jnp.concatenate([gsum, gmax])[None, :]
    z = jnp.maximum(
        jnp.dot(g, wc1_ref[...], preferred_element_type=jnp.float32)
        + bc1_ref[...], 0.0)
    z = (z * float(1.0 / (1.0 + 1e-5) ** 0.5)) * gamma_ref[...] + beta_ref[...]
    zo = jnp.dot(z, wc2_ref[...], preferred_element_type=jnp.float32) \
        + bc2_ref[...]
    out_ref[...] = 1.0 / (1.0 + jnp.exp(-zo))


def _tc_call(fn, out_shapes):
    return pl.pallas_call(fn, out_shape=out_shapes)


def kernel(x, edge_index, W0, b0, Wres0, bres0, W1, b1, Wres1, bres1,
           Wg, bg, gamma, beta, Wc1, bc1, Wc2, bc2):
    e = edge_index.shape[1]
    blk = 8
    step = NW * CH
    nch = (e + step - 1) // step
    nch = ((nch + 2 * blk - 1) // (2 * blk)) * (2 * blk)
    nblk = nch // blk
    epad = nch * step

    src = edge_index[0].astype(jnp.int32)
    dst = edge_index[1].astype(jnp.int32)
    pad_idx = (N + jnp.arange(epad - e, dtype=jnp.int32) % PADROWS)
    src3 = jnp.concatenate([src, pad_idx]).reshape(NW, nch, CH)
    dst3 = jnp.concatenate([dst, pad_idx]).reshape(NW, nch, CH)
    eidx = jnp.stack([src3, dst3], axis=2).reshape(NW, 2 * nch, CH)

    zeros_nd = jnp.zeros((NPAD, D), jnp.float32)
    zeros_flat = jnp.zeros((NPAD,), jnp.float32)

    degp = _sc_degrees(nch)(eidx, zeros_flat)

    f32 = jnp.float32
    xs, ns, nd = _tc_call(_tc_prep, [
        jax.ShapeDtypeStruct((NPAD, D), f32),
        jax.ShapeDtypeStruct((NPAD, 1), f32),
        jax.ShapeDtypeStruct((NPAD, 1), f32),
    ])(degp, x)

    agg_fn = _sc_aggregate(nch, nblk, blk)
    p0 = agg_fn(xs, eidx, zeros_nd)

    h1, h1s = _tc_call(_tc_layer, [
        jax.ShapeDtypeStruct((NPAD, H), f32),
        jax.ShapeDtypeStruct((NPAD, H), f32),
    ])(p0, x, ns, nd, W0, b0[None, :], Wres0, bres0[None, :])

    p1 = agg_fn(h1s, eidx, zeros_nd)

    (out,) = _tc_call(_tc_final, [jax.ShapeDtypeStruct((1, 1), f32)])(
        p1, h1, nd, W1, b1[None, :], Wres1, bres1[None, :],
        Wg, bg[None, :], gamma[None, :], beta[None, :],
        Wc1, bc1[None, :], Wc2, bc2[None, :])
    return out
